# baseline (speedup 1.0000x reference)
import functools

import jax
import jax.numpy as jnp
from jax import lax
from jax.experimental import pallas as pl
from jax.experimental.pallas import tpu as pltpu
from jax.experimental.pallas import tpu_sc as plsc

N_NODES = 10000
D = 128
NC = 2
NS = 16
LANES = 16
NW = NC * NS

CHUNK = 50
NPAD = 10240
ROWS_PER_SUB = NPAD // NS
NB = 4

_mesh = plsc.VectorSubcoreMesh(core_axis_name="c", subcore_axis_name="s")
_sc_params = pltpu.CompilerParams(use_tc_tiling_on_sc=False)


def _sc_degrees(src3, dst3):
    n_chunks = src3.shape[1]

    @functools.partial(
        pl.kernel,
        out_type=jax.ShapeDtypeStruct((NC, 2, NPAD, LANES), jnp.float32),
        mesh=_mesh,
        scratch_types=[
            pltpu.VMEM((n_chunks, CHUNK), jnp.int32),
            pltpu.VMEM((n_chunks, CHUNK), jnp.int32),
            pltpu.VMEM((CHUNK, LANES), jnp.float32),
            pltpu.VMEM((ROWS_PER_SUB, LANES), jnp.float32),
            pltpu.VMEM_SHARED((NPAD, LANES), jnp.float32),
            pltpu.VMEM_SHARED((NPAD, LANES), jnp.float32),
            pltpu.SemaphoreType.DMA,
        ],
        compiler_params=_sc_params,
    )
    def k(src_hbm, dst_hbm, out_hbm, sidx_v, didx_v, ones_v, zero_v,
          acc0, acc1, ssem):
        cid = lax.axis_index("c")
        sid = lax.axis_index("s")
        wid = cid * NS + sid

        def fill_ones(i, _):
            ones_v[i, :] = jnp.full((LANES,), 1.0, jnp.float32)
            return 0

        lax.fori_loop(0, CHUNK, fill_ones, 0)

        def fill_zero(i, _):
            zero_v[i, :] = jnp.zeros((LANES,), jnp.float32)
            return 0

        lax.fori_loop(0, ROWS_PER_SUB, fill_zero, 0)

        pltpu.sync_copy(src_hbm.at[wid], sidx_v)
        pltpu.sync_copy(dst_hbm.at[wid], didx_v)

        row0 = sid * ROWS_PER_SUB
        pltpu.sync_copy(zero_v, acc0.at[pl.ds(row0, ROWS_PER_SUB)])
        pltpu.sync_copy(zero_v, acc1.at[pl.ds(row0, ROWS_PER_SUB)])
        plsc.subcore_barrier()

        def fire(c, _):
            pltpu.async_copy(ones_v, acc0.at[sidx_v.at[c]], ssem, add=True)
            pltpu.async_copy(ones_v, acc1.at[didx_v.at[c]], ssem, add=True)
            return 0

        lax.fori_loop(0, n_chunks, fire, 0)

        def drain(c, _):
            pltpu.make_async_copy(ones_v, acc0.at[sidx_v.at[c]], ssem).wait()
            pltpu.make_async_copy(ones_v, acc1.at[didx_v.at[c]], ssem).wait()
            return 0

        lax.fori_loop(0, n_chunks, drain, 0)
        plsc.subcore_barrier()

        pltpu.sync_copy(acc0.at[pl.ds(row0, ROWS_PER_SUB)],
                        out_hbm.at[cid, 0, pl.ds(row0, ROWS_PER_SUB)])
        pltpu.sync_copy(acc1.at[pl.ds(row0, ROWS_PER_SUB)],
                        out_hbm.at[cid, 1, pl.ds(row0, ROWS_PER_SUB)])

    return k(src3, dst3)


def _sc_propagate(q, src3, dst3):
    n_chunks = src3.shape[1]
    assert n_chunks >= 8

    @functools.partial(
        pl.kernel,
        out_type=jax.ShapeDtypeStruct((NC, NPAD, D), jnp.float32),
        mesh=_mesh,
        scratch_types=[
            pltpu.VMEM((n_chunks, CHUNK), jnp.int32),
            pltpu.VMEM((n_chunks, CHUNK), jnp.int32),
            pltpu.VMEM((NB, CHUNK, D), jnp.float32),
            pltpu.VMEM_SHARED((NPAD, D), jnp.float32),
            [pltpu.SemaphoreType.DMA] * NB,
            [pltpu.SemaphoreType.DMA] * NB,
        ],
        compiler_params=_sc_params,
    )
    def k(q_hbm, src_hbm, dst_hbm, out_hbm, sidx_v, didx_v, rows_v,
          acc, gsems, ssems):
        cid = lax.axis_index("c")
        sid = lax.axis_index("s")
        wid = cid * NS + sid

        def fill_zero(i, _):
            for j in range(D // LANES):
                rows_v[0, i, pl.ds(j * LANES, LANES)] = jnp.zeros(
                    (LANES,), jnp.float32)
            return 0

        lax.fori_loop(0, 16, fill_zero, 0)

        pltpu.sync_copy(src_hbm.at[wid], sidx_v)
        pltpu.sync_copy(dst_hbm.at[wid], didx_v)

        row0 = sid * ROWS_PER_SUB
        for t in range(ROWS_PER_SUB // 16):
            pltpu.sync_copy(rows_v.at[0, pl.ds(0, 16)],
                            acc.at[pl.ds(row0 + t * 16, 16)])
        plsc.subcore_barrier()

        def gather(c, b):
            pltpu.async_copy(q_hbm.at[sidx_v.at[c]], rows_v.at[b], gsems[b])

        def wait_gather(c, b):
            pltpu.make_async_copy(q_hbm.at[sidx_v.at[c]], rows_v.at[b],
                                  gsems[b]).wait()

        def scatter(c, b):
            pltpu.async_copy(rows_v.at[b], acc.at[didx_v.at[c]], ssems[b],
                             add=True)

        def wait_scatter(c, b):
            pltpu.make_async_copy(rows_v.at[b], acc.at[didx_v.at[c]],
                                  ssems[b]).wait()

        for c0 in range(NB):
            gather(c0, c0)
        for c0 in range(2):
            wait_gather(c0, c0)
            scatter(c0, c0)

        gd = NB - 2
        ngroups = (n_chunks - 2 - gd) // NB
        c_tail = 2 + NB * ngroups

        def group(o, _):
            for j in range(NB):
                c = NB * o + 2 + j
                b = (2 + j) % NB
                wait_gather(c, b)
                scatter(c, b)
                wait_scatter(c - 2, j)
                gather(c + gd, j)
            return 0

        lax.fori_loop(0, ngroups, group, 0)
        for c in range(c_tail, n_chunks):
            b = c % NB
            wait_gather(c, b)
            scatter(c, b)
            wait_scatter(c - 2, (c - 2) % NB)
            if c + gd < n_chunks:
                gather(c + gd, (c + gd) % NB)
        wait_scatter(n_chunks - 2, (n_chunks - 2) % NB)
        wait_scatter(n_chunks - 1, (n_chunks - 1) % NB)
        plsc.subcore_barrier()

        pltpu.sync_copy(acc.at[pl.ds(row0, ROWS_PER_SUB)],
                        out_hbm.at[cid, pl.ds(row0, ROWS_PER_SUB)])

    return k(q, src3, dst3)


_BLK = 2000


def _norms_from(dp):
    deg_out = dp[0, 0, :, 0] + dp[1, 0, :, 0]
    deg_in = dp[0, 1, :, 0] + dp[1, 1, :, 0]
    ns = lax.rsqrt(jnp.maximum(deg_out, 1.0))
    nd = lax.rsqrt(jnp.maximum(deg_in, 1.0))
    return ns, nd


def _tc_prep_body(x_ref, dp_ref, o_ref):
    ns, _ = _norms_from(dp_ref[...])
    o_ref[...] = x_ref[...] * ns[:, None]


def _tc_prep(x, degp):
    return pl.pallas_call(
        _tc_prep_body,
        out_shape=jax.ShapeDtypeStruct((N_NODES, D), jnp.float32),
        grid=(N_NODES // _BLK,),
        in_specs=[
            pl.BlockSpec((_BLK, D), lambda i: (i, 0)),
            pl.BlockSpec((NC, 2, _BLK, LANES), lambda i: (0, 0, i, 0)),
        ],
        out_specs=pl.BlockSpec((_BLK, D), lambda i: (i, 0)),
    )(x, degp)


def _tc_layer_body(last, p_ref, dp_ref, w_ref, b_ref, o_ref):
    ns, nd = _norms_from(dp_ref[...])
    s = (p_ref[0] + p_ref[1]) * nd[:, None]
    h = jnp.dot(s, w_ref[...], preferred_element_type=jnp.float32)
    h = jnp.maximum(h + b_ref[...], 0.0)
    if not last:
        h = h * ns[:, None]
    o_ref[...] = h


def _tc_layer(p, degp, w, b2d, last):
    return pl.pallas_call(
        functools.partial(_tc_layer_body, last),
        out_shape=jax.ShapeDtypeStruct((N_NODES, D), jnp.float32),
        grid=(N_NODES // _BLK,),
        in_specs=[
            pl.BlockSpec((NC, _BLK, D), lambda i: (0, i, 0)),
            pl.BlockSpec((NC, 2, _BLK, LANES), lambda i: (0, 0, i, 0)),
            pl.BlockSpec((D, D), lambda i: (0, 0)),
            pl.BlockSpec((1, D), lambda i: (0, 0)),
        ],
        out_specs=pl.BlockSpec((_BLK, D), lambda i: (i, 0)),
    )(p, degp, w, b2d)


def kernel(x, edge_index, W0, b0, W1, b1, W2, b2, W3, b3, W4, b4):
    ei = edge_index.astype(jnp.int32)
    src3 = ei[0].reshape(NW, -1, CHUNK)
    dst3 = ei[1].reshape(NW, -1, CHUNK)
    degp = _sc_degrees(src3, dst3)
    q = _tc_prep(x, degp)
    Ws = [W0, W1, W2, W3, W4]
    bs = [b0, b1, b2, b3, b4]
    for l in range(5):
        p = _sc_propagate(q, src3, dst3)
        q = _tc_layer(p, degp, Ws[l], bs[l].reshape(1, D), last=(l == 4))
    return q


# speedup vs baseline: 1.1786x; 1.1786x over previous
import functools

import jax
import jax.numpy as jnp
from jax import lax
from jax.experimental import pallas as pl
from jax.experimental.pallas import tpu as pltpu
from jax.experimental.pallas import tpu_sc as plsc

N_NODES = 10000
D = 128
NC = 2
NS = 16
LANES = 16
NW = NC * NS

CHUNK = 40
NPAD = 10240
ROWS_PER_SUB = NPAD // NS
NB = 5

_mesh = plsc.VectorSubcoreMesh(core_axis_name="c", subcore_axis_name="s")
_sc_params = pltpu.CompilerParams(use_tc_tiling_on_sc=False)


def _sc_degrees(src3, dst3):
    n_chunks = src3.shape[1]

    @functools.partial(
        pl.kernel,
        out_type=jax.ShapeDtypeStruct((NC, 2, NPAD, LANES), jnp.float32),
        mesh=_mesh,
        scratch_types=[
            pltpu.VMEM((n_chunks, CHUNK), jnp.int32),
            pltpu.VMEM((n_chunks, CHUNK), jnp.int32),
            pltpu.VMEM((CHUNK, LANES), jnp.float32),
            pltpu.VMEM((ROWS_PER_SUB, LANES), jnp.float32),
            pltpu.VMEM_SHARED((NPAD, LANES), jnp.float32),
            pltpu.VMEM_SHARED((NPAD, LANES), jnp.float32),
            pltpu.SemaphoreType.DMA,
        ],
        compiler_params=_sc_params,
    )
    def k(src_hbm, dst_hbm, out_hbm, sidx_v, didx_v, ones_v, zero_v,
          acc0, acc1, ssem):
        cid = lax.axis_index("c")
        sid = lax.axis_index("s")
        wid = cid * NS + sid

        def fill_ones(i, _):
            ones_v[i, :] = jnp.full((LANES,), 1.0, jnp.float32)
            return 0

        lax.fori_loop(0, CHUNK, fill_ones, 0)

        def fill_zero(i, _):
            zero_v[i, :] = jnp.zeros((LANES,), jnp.float32)
            return 0

        lax.fori_loop(0, ROWS_PER_SUB, fill_zero, 0)

        pltpu.sync_copy(src_hbm.at[wid], sidx_v)
        pltpu.sync_copy(dst_hbm.at[wid], didx_v)

        row0 = sid * ROWS_PER_SUB
        pltpu.sync_copy(zero_v, acc0.at[pl.ds(row0, ROWS_PER_SUB)])
        pltpu.sync_copy(zero_v, acc1.at[pl.ds(row0, ROWS_PER_SUB)])
        plsc.subcore_barrier()

        def fire(c, _):
            pltpu.async_copy(ones_v, acc0.at[sidx_v.at[c]], ssem, add=True)
            pltpu.async_copy(ones_v, acc1.at[didx_v.at[c]], ssem, add=True)
            return 0

        lax.fori_loop(0, n_chunks, fire, 0)

        def drain(c, _):
            pltpu.make_async_copy(ones_v, acc0.at[sidx_v.at[c]], ssem).wait()
            pltpu.make_async_copy(ones_v, acc1.at[didx_v.at[c]], ssem).wait()
            return 0

        lax.fori_loop(0, n_chunks, drain, 0)
        plsc.subcore_barrier()

        pltpu.sync_copy(acc0.at[pl.ds(row0, ROWS_PER_SUB)],
                        out_hbm.at[cid, 0, pl.ds(row0, ROWS_PER_SUB)])
        pltpu.sync_copy(acc1.at[pl.ds(row0, ROWS_PER_SUB)],
                        out_hbm.at[cid, 1, pl.ds(row0, ROWS_PER_SUB)])

    return k(src3, dst3)


def _sc_propagate(q, src3, dst3):
    n_chunks = src3.shape[1]
    assert n_chunks >= 8

    @functools.partial(
        pl.kernel,
        out_type=jax.ShapeDtypeStruct((NC, NPAD, D), jnp.float32),
        mesh=_mesh,
        scratch_types=[
            pltpu.VMEM((n_chunks, CHUNK), jnp.int32),
            pltpu.VMEM((n_chunks, CHUNK), jnp.int32),
            pltpu.VMEM((NB, CHUNK, D), jnp.float32),
            pltpu.VMEM_SHARED((NPAD, D), jnp.float32),
            [pltpu.SemaphoreType.DMA] * NB,
            [pltpu.SemaphoreType.DMA] * NB,
        ],
        compiler_params=_sc_params,
    )
    def k(q_hbm, src_hbm, dst_hbm, out_hbm, sidx_v, didx_v, rows_v,
          acc, gsems, ssems):
        cid = lax.axis_index("c")
        sid = lax.axis_index("s")
        wid = cid * NS + sid

        def fill_zero(i, _):
            for j in range(D // LANES):
                rows_v[0, i, pl.ds(j * LANES, LANES)] = jnp.zeros(
                    (LANES,), jnp.float32)
            return 0

        lax.fori_loop(0, 16, fill_zero, 0)

        pltpu.sync_copy(src_hbm.at[wid], sidx_v)
        pltpu.sync_copy(dst_hbm.at[wid], didx_v)

        row0 = sid * ROWS_PER_SUB
        for t in range(ROWS_PER_SUB // 16):
            pltpu.sync_copy(rows_v.at[0, pl.ds(0, 16)],
                            acc.at[pl.ds(row0 + t * 16, 16)])
        plsc.subcore_barrier()

        def gather(c, b):
            pltpu.async_copy(q_hbm.at[sidx_v.at[c]], rows_v.at[b], gsems[b])

        def wait_gather(c, b):
            pltpu.make_async_copy(q_hbm.at[sidx_v.at[c]], rows_v.at[b],
                                  gsems[b]).wait()

        def scatter(c, b):
            pltpu.async_copy(rows_v.at[b], acc.at[didx_v.at[c]], ssems[b],
                             add=True)

        def wait_scatter(c, b):
            pltpu.make_async_copy(rows_v.at[b], acc.at[didx_v.at[c]],
                                  ssems[b]).wait()

        for c0 in range(NB):
            gather(c0, c0)
        for c0 in range(2):
            wait_gather(c0, c0)
            scatter(c0, c0)

        gd = NB - 2
        ngroups = (n_chunks - 2 - gd) // NB
        c_tail = 2 + NB * ngroups

        def group(o, _):
            for j in range(NB):
                c = NB * o + 2 + j
                b = (2 + j) % NB
                wait_gather(c, b)
                scatter(c, b)
                wait_scatter(c - 2, j)
                gather(c + gd, j)
            return 0

        lax.fori_loop(0, ngroups, group, 0)
        for c in range(c_tail, n_chunks):
            b = c % NB
            wait_gather(c, b)
            scatter(c, b)
            wait_scatter(c - 2, (c - 2) % NB)
            if c + gd < n_chunks:
                gather(c + gd, (c + gd) % NB)
        wait_scatter(n_chunks - 2, (n_chunks - 2) % NB)
        wait_scatter(n_chunks - 1, (n_chunks - 1) % NB)
        plsc.subcore_barrier()

        pltpu.sync_copy(acc.at[pl.ds(row0, ROWS_PER_SUB)],
                        out_hbm.at[cid, pl.ds(row0, ROWS_PER_SUB)])

    return k(q, src3, dst3)


_BLK = 2000


def _norms_from(dp):
    deg_out = dp[0, 0, :, 0] + dp[1, 0, :, 0]
    deg_in = dp[0, 1, :, 0] + dp[1, 1, :, 0]
    ns = lax.rsqrt(jnp.maximum(deg_out, 1.0))
    nd = lax.rsqrt(jnp.maximum(deg_in, 1.0))
    return ns, nd


def _tc_prep_body(x_ref, dp_ref, o_ref):
    ns, _ = _norms_from(dp_ref[...])
    o_ref[...] = x_ref[...] * ns[:, None]


def _tc_prep(x, degp):
    return pl.pallas_call(
        _tc_prep_body,
        out_shape=jax.ShapeDtypeStruct((N_NODES, D), jnp.float32),
        grid=(N_NODES // _BLK,),
        in_specs=[
            pl.BlockSpec((_BLK, D), lambda i: (i, 0)),
            pl.BlockSpec((NC, 2, _BLK, LANES), lambda i: (0, 0, i, 0)),
        ],
        out_specs=pl.BlockSpec((_BLK, D), lambda i: (i, 0)),
    )(x, degp)


def _tc_layer_body(last, p_ref, dp_ref, w_ref, b_ref, o_ref):
    ns, nd = _norms_from(dp_ref[...])
    s = (p_ref[0] + p_ref[1]) * nd[:, None]
    h = jnp.dot(s, w_ref[...], preferred_element_type=jnp.float32)
    h = jnp.maximum(h + b_ref[...], 0.0)
    if not last:
        h = h * ns[:, None]
    o_ref[...] = h


def _tc_layer(p, degp, w, b2d, last):
    return pl.pallas_call(
        functools.partial(_tc_layer_body, last),
        out_shape=jax.ShapeDtypeStruct((N_NODES, D), jnp.float32),
        grid=(N_NODES // _BLK,),
        in_specs=[
            pl.BlockSpec((NC, _BLK, D), lambda i: (0, i, 0)),
            pl.BlockSpec((NC, 2, _BLK, LANES), lambda i: (0, 0, i, 0)),
            pl.BlockSpec((D, D), lambda i: (0, 0)),
            pl.BlockSpec((1, D), lambda i: (0, 0)),
        ],
        out_specs=pl.BlockSpec((_BLK, D), lambda i: (i, 0)),
    )(p, degp, w, b2d)


def kernel(x, edge_index, W0, b0, W1, b1, W2, b2, W3, b3, W4, b4):
    ei = edge_index.astype(jnp.int32)
    src3 = ei[0].reshape(NW, -1, CHUNK)
    dst3 = ei[1].reshape(NW, -1, CHUNK)
    degp = _sc_degrees(src3, dst3)
    q = _tc_prep(x, degp)
    Ws = [W0, W1, W2, W3, W4]
    bs = [b0, b1, b2, b3, b4]
    for l in range(5):
        p = _sc_propagate(q, src3, dst3)
        q = _tc_layer(p, degp, Ws[l], bs[l].reshape(1, D), last=(l == 4))
    return q


# speedup vs baseline: 1.2971x; 1.1005x over previous
import functools

import jax
import jax.numpy as jnp
from jax import lax
from jax.experimental import pallas as pl
from jax.experimental.pallas import tpu as pltpu
from jax.experimental.pallas import tpu_sc as plsc

N_NODES = 10000
D = 128
NC = 2
NS = 16
LANES = 16
NW = NC * NS

CHUNK = 40
NPAD = 10240
ROWS_PER_SUB = NPAD // NS
NB = 7

_mesh = plsc.VectorSubcoreMesh(core_axis_name="c", subcore_axis_name="s")
_sc_params = pltpu.CompilerParams(use_tc_tiling_on_sc=False)


def _sc_degrees(src3, dst3):
    n_chunks = src3.shape[1]

    @functools.partial(
        pl.kernel,
        out_type=jax.ShapeDtypeStruct((NC, 2, NPAD, LANES), jnp.float32),
        mesh=_mesh,
        scratch_types=[
            pltpu.VMEM((n_chunks, CHUNK), jnp.int32),
            pltpu.VMEM((n_chunks, CHUNK), jnp.int32),
            pltpu.VMEM((CHUNK, LANES), jnp.float32),
            pltpu.VMEM((ROWS_PER_SUB, LANES), jnp.float32),
            pltpu.VMEM_SHARED((NPAD, LANES), jnp.float32),
            pltpu.VMEM_SHARED((NPAD, LANES), jnp.float32),
            pltpu.SemaphoreType.DMA,
        ],
        compiler_params=_sc_params,
    )
    def k(src_hbm, dst_hbm, out_hbm, sidx_v, didx_v, ones_v, zero_v,
          acc0, acc1, ssem):
        cid = lax.axis_index("c")
        sid = lax.axis_index("s")
        wid = cid * NS + sid

        def fill_ones(i, _):
            ones_v[i, :] = jnp.full((LANES,), 1.0, jnp.float32)
            return 0

        lax.fori_loop(0, CHUNK, fill_ones, 0)

        def fill_zero(i, _):
            zero_v[i, :] = jnp.zeros((LANES,), jnp.float32)
            return 0

        lax.fori_loop(0, ROWS_PER_SUB, fill_zero, 0)

        pltpu.sync_copy(src_hbm.at[wid], sidx_v)
        pltpu.sync_copy(dst_hbm.at[wid], didx_v)

        row0 = sid * ROWS_PER_SUB
        pltpu.sync_copy(zero_v, acc0.at[pl.ds(row0, ROWS_PER_SUB)])
        pltpu.sync_copy(zero_v, acc1.at[pl.ds(row0, ROWS_PER_SUB)])
        plsc.subcore_barrier()

        def fire(c, _):
            pltpu.async_copy(ones_v, acc0.at[sidx_v.at[c]], ssem, add=True)
            pltpu.async_copy(ones_v, acc1.at[didx_v.at[c]], ssem, add=True)
            return 0

        lax.fori_loop(0, n_chunks, fire, 0)

        def drain(c, _):
            pltpu.make_async_copy(ones_v, acc0.at[sidx_v.at[c]], ssem).wait()
            pltpu.make_async_copy(ones_v, acc1.at[didx_v.at[c]], ssem).wait()
            return 0

        lax.fori_loop(0, n_chunks, drain, 0)
        plsc.subcore_barrier()

        pltpu.sync_copy(acc0.at[pl.ds(row0, ROWS_PER_SUB)],
                        out_hbm.at[cid, 0, pl.ds(row0, ROWS_PER_SUB)])
        pltpu.sync_copy(acc1.at[pl.ds(row0, ROWS_PER_SUB)],
                        out_hbm.at[cid, 1, pl.ds(row0, ROWS_PER_SUB)])

    return k(src3, dst3)


def _sc_propagate(q, src3, dst3):
    n_chunks = src3.shape[1]
    n_half = n_chunks // 2
    assert n_half * 2 == n_chunks and n_half >= NB

    @functools.partial(
        pl.kernel,
        out_type=jax.ShapeDtypeStruct((NC, NPAD, D), jnp.float32),
        mesh=_mesh,
        scratch_types=[
            pltpu.VMEM((n_half, CHUNK), jnp.int32),
            pltpu.VMEM((n_half, CHUNK), jnp.int32),
            pltpu.VMEM((NB, CHUNK, D), jnp.float32),
            pltpu.VMEM_SHARED((NPAD, D), jnp.float32),
            [pltpu.SemaphoreType.DMA] * NB,
            [pltpu.SemaphoreType.DMA] * NB,
        ],
        compiler_params=_sc_params,
    )
    def k(q_hbm, src_hbm, dst_hbm, out_hbm, sidx_v, didx_v, rows_v,
          acc, gsems, ssems):
        cid = lax.axis_index("c")
        sid = lax.axis_index("s")
        wid = cid * NS + sid

        def fill_zero(i, _):
            for j in range(D // LANES):
                rows_v[0, i, pl.ds(j * LANES, LANES)] = jnp.zeros(
                    (LANES,), jnp.float32)
            return 0

        lax.fori_loop(0, 16, fill_zero, 0)

        row0 = sid * ROWS_PER_SUB
        for t in range(ROWS_PER_SUB // 16):
            pltpu.sync_copy(rows_v.at[0, pl.ds(0, 16)],
                            acc.at[pl.ds(row0 + t * 16, 16)])

        def gather(c, b):
            pltpu.async_copy(q_hbm.at[sidx_v.at[c]], rows_v.at[b], gsems[b])

        def wait_gather(c, b):
            pltpu.make_async_copy(q_hbm.at[sidx_v.at[c]], rows_v.at[b],
                                  gsems[b]).wait()

        def scatter(c, b):
            pltpu.async_copy(rows_v.at[b], acc.at[didx_v.at[c]], ssems[b],
                             add=True)

        def wait_scatter(c, b):
            pltpu.make_async_copy(rows_v.at[b], acc.at[didx_v.at[c]],
                                  ssems[b]).wait()

        gd = NB - 2

        def run_half(h):
            pltpu.sync_copy(src_hbm.at[wid, pl.ds(h * n_half, n_half)], sidx_v)
            pltpu.sync_copy(dst_hbm.at[wid, pl.ds(h * n_half, n_half)], didx_v)
            if h == 0:
                plsc.subcore_barrier()

            for c0 in range(NB):
                gather(c0, c0)
            for c0 in range(2):
                wait_gather(c0, c0)
                scatter(c0, c0)

            ngroups = (n_half - 2 - gd) // NB
            c_tail = 2 + NB * ngroups

            def group(o, _):
                for j in range(NB):
                    c = NB * o + 2 + j
                    b = (2 + j) % NB
                    wait_gather(c, b)
                    scatter(c, b)
                    wait_scatter(c - 2, j)
                    gather(c + gd, j)
                return 0

            lax.fori_loop(0, ngroups, group, 0)
            for c in range(c_tail, n_half):
                b = c % NB
                wait_gather(c, b)
                scatter(c, b)
                wait_scatter(c - 2, (c - 2) % NB)
                if c + gd < n_half:
                    gather(c + gd, (c + gd) % NB)
            wait_scatter(n_half - 2, (n_half - 2) % NB)
            wait_scatter(n_half - 1, (n_half - 1) % NB)

        run_half(0)
        run_half(1)
        plsc.subcore_barrier()

        pltpu.sync_copy(acc.at[pl.ds(row0, ROWS_PER_SUB)],
                        out_hbm.at[cid, pl.ds(row0, ROWS_PER_SUB)])

    return k(q, src3, dst3)


_BLK = 2000


def _norms_from(dp):
    deg_out = dp[0, 0, :, 0] + dp[1, 0, :, 0]
    deg_in = dp[0, 1, :, 0] + dp[1, 1, :, 0]
    ns = lax.rsqrt(jnp.maximum(deg_out, 1.0))
    nd = lax.rsqrt(jnp.maximum(deg_in, 1.0))
    return ns, nd


def _tc_prep_body(x_ref, dp_ref, o_ref):
    ns, _ = _norms_from(dp_ref[...])
    o_ref[...] = x_ref[...] * ns[:, None]


def _tc_prep(x, degp):
    return pl.pallas_call(
        _tc_prep_body,
        out_shape=jax.ShapeDtypeStruct((N_NODES, D), jnp.float32),
        grid=(N_NODES // _BLK,),
        in_specs=[
            pl.BlockSpec((_BLK, D), lambda i: (i, 0)),
            pl.BlockSpec((NC, 2, _BLK, LANES), lambda i: (0, 0, i, 0)),
        ],
        out_specs=pl.BlockSpec((_BLK, D), lambda i: (i, 0)),
    )(x, degp)


def _tc_layer_body(last, p_ref, dp_ref, w_ref, b_ref, o_ref):
    ns, nd = _norms_from(dp_ref[...])
    s = (p_ref[0] + p_ref[1]) * nd[:, None]
    h = jnp.dot(s, w_ref[...], preferred_element_type=jnp.float32)
    h = jnp.maximum(h + b_ref[...], 0.0)
    if not last:
        h = h * ns[:, None]
    o_ref[...] = h


def _tc_layer(p, degp, w, b2d, last):
    return pl.pallas_call(
        functools.partial(_tc_layer_body, last),
        out_shape=jax.ShapeDtypeStruct((N_NODES, D), jnp.float32),
        grid=(N_NODES // _BLK,),
        in_specs=[
            pl.BlockSpec((NC, _BLK, D), lambda i: (0, i, 0)),
            pl.BlockSpec((NC, 2, _BLK, LANES), lambda i: (0, 0, i, 0)),
            pl.BlockSpec((D, D), lambda i: (0, 0)),
            pl.BlockSpec((1, D), lambda i: (0, 0)),
        ],
        out_specs=pl.BlockSpec((_BLK, D), lambda i: (i, 0)),
    )(p, degp, w, b2d)


def kernel(x, edge_index, W0, b0, W1, b1, W2, b2, W3, b3, W4, b4):
    ei = edge_index.astype(jnp.int32)
    src3 = ei[0].reshape(NW, -1, CHUNK)
    dst3 = ei[1].reshape(NW, -1, CHUNK)
    degp = _sc_degrees(src3, dst3)
    q = _tc_prep(x, degp)
    Ws = [W0, W1, W2, W3, W4]
    bs = [b0, b1, b2, b3, b4]
    for l in range(5):
        p = _sc_propagate(q, src3, dst3)
        q = _tc_layer(p, degp, Ws[l], bs[l].reshape(1, D), last=(l == 4))
    return q
